# baseline (speedup 1.0000x reference)
import functools

import jax
import jax.numpy as jnp
from jax import lax
from jax.experimental import pallas as pl
from jax.experimental.pallas import tpu as pltpu
from jax.experimental.pallas import tpu_sc as plsc

NC = 2
NS = 16
NW = NC * NS
B = 128
BLK = 128

_MESH = functools.partial(
    plsc.VectorSubcoreMesh,
    core_axis_name="c", subcore_axis_name="s", num_cores=NC, num_subcores=NS,
)


def _worker_id():
    return lax.axis_index("s") * NC + lax.axis_index("c")


WD = 128


def _make_pass_a(N, D, NP, EPT):
    RPS = NP // NS
    NBX = NP // B
    TX = (NBX + NW - 1) // NW
    NBE = EPT // B

    NOUT = 8

    @functools.partial(
        pl.kernel,
        out_type=(
            jax.ShapeDtypeStruct((NP, D), jnp.float32),
            jax.ShapeDtypeStruct((NC, NP, WD), jnp.float32),
        ),
        mesh=_MESH(),
        scratch_types=[
            pltpu.VMEM((B,), jnp.int32),
            pltpu.VMEM((B, D), jnp.float32),
            pltpu.VMEM((B, WD), jnp.float32),
            pltpu.VMEM_SHARED((NP, WD), jnp.float32),
            pltpu.SemaphoreType.DMA,
        ],
    )
    def pass_a(emb_h, xind_h, col_h, ones_h, zer8_h, x_h, degp_h,
               idxv, rows, ones_v, deg_acc, sem):
        c = lax.axis_index("c")
        s = lax.axis_index("s")
        w = _worker_id()
        pltpu.sync_copy(zer8_h, deg_acc.at[pl.ds(s * RPS, RPS)])
        pltpu.sync_copy(ones_h, ones_v)
        plsc.subcore_barrier()

        def xgather(t, carry):
            bid = w + NW * t

            @pl.when(bid < NBX)
            def _():
                pltpu.sync_copy(xind_h.at[pl.ds(bid * B, B)], idxv)
                pltpu.async_copy(emb_h.at[idxv], rows, sem).wait()
                pltpu.sync_copy(rows, x_h.at[pl.ds(bid * B, B)])
            return carry

        lax.fori_loop(0, TX, xgather, 0)

        def deg_scatter(bi, carry):
            base = w * EPT + bi * B
            pltpu.sync_copy(col_h.at[pl.ds(base, B)], idxv)
            pltpu.sync_copy(ones_v, deg_acc.at[idxv], add=True)
            return carry

        lax.fori_loop(0, NBE, deg_scatter, 0)
        plsc.subcore_barrier()
        pltpu.sync_copy(deg_acc.at[pl.ds(s * RPS, RPS)],
                        degp_h.at[c, pl.ds(s * RPS, RPS)])

    return pass_a


NBUF = 2


def _make_prop(D, NP, EPT):
    RPS = NP // NS
    NBE = EPT // B
    assert NBE % 2 == 0
    NB2 = NBE // 2

    @functools.partial(
        pl.kernel,
        out_type=jax.ShapeDtypeStruct((NC, NP, D), jnp.float32),
        mesh=_MESH(),
        scratch_types=[
            pltpu.VMEM((NBE, B), jnp.int32),
            pltpu.VMEM((B,), jnp.int32),
            pltpu.VMEM((B,), jnp.int32),
            pltpu.VMEM((B, D), jnp.float32),
            pltpu.VMEM((B, D), jnp.float32),
            pltpu.VMEM_SHARED((NP, D), jnp.float32),
            pltpu.SemaphoreType.DMA,
            pltpu.SemaphoreType.DMA,
            pltpu.SemaphoreType.DMA,
        ],
    )
    def prop(z_h, row_h, col_h, sp_h, coli, rv0, rv1, rows0, rows1,
             acc, isem, gsem, ssem):
        c = lax.axis_index("c")
        s = lax.axis_index("s")
        w = _worker_id()
        rv = (rv0, rv1)
        rows = (rows0, rows1)
        pltpu.sync_copy(col_h.at[w], coli)
        pltpu.sync_copy(z_h.at[pl.ds(s * RPS, RPS)],
                        acc.at[pl.ds(s * RPS, RPS)])
        plsc.subcore_barrier()

        def fire_idx(i, p):
            pltpu.async_copy(row_h.at[pl.ds(w * EPT + i * B, B)], rv[p], isem)

        def wait_idx():
            pltpu.make_async_copy(row_h.at[pl.ds(0, B)], rv0, isem).wait()

        def fire_gather(p):
            pltpu.async_copy(z_h.at[rv[p]], rows[p], gsem)

        def wait_gather():
            pltpu.make_async_copy(z_h.at[rv0], rows0, gsem).wait()

        def fire_scatter(i, p):
            pltpu.async_copy(rows[p], acc.at[coli.at[i]], ssem, add=True)

        def wait_scatter():
            pltpu.make_async_copy(rows0, acc.at[coli.at[0]], ssem).wait()

        fire_idx(0, 0)
        fire_idx(1, 1)
        wait_idx()
        fire_gather(0)

        def pair(g, carry):
            for jj in range(2):
                i = 2 * g + jj
                wait_gather()
                fire_scatter(i, jj)

                @pl.when(i + 2 < NBE)
                def _():
                    fire_idx(i + 2, jj)

                @pl.when(i >= 1)
                def _():
                    wait_scatter()

                @pl.when(i + 1 < NBE)
                def _():
                    wait_idx()
                    fire_gather(1 - jj)
            return carry

        lax.fori_loop(0, NB2, pair, 0)
        wait_scatter()
        plsc.subcore_barrier()
        pltpu.sync_copy(acc.at[pl.ds(s * RPS, RPS)],
                        sp_h.at[c, pl.ds(s * RPS, RPS)])

    return prop


def _deg_of(degp_ref):
    return degp_ref[0, :, 0] + degp_ref[1, :, 0] + 1.0


def _scale_z0(degp_ref, x_ref, z_ref):
    r = lax.rsqrt(_deg_of(degp_ref))
    z_ref[...] = x_ref[...] * r[:, None]


def _combine_mid(degp_ref, sp_ref, z0_ref, z1_ref):
    dinv = 1.0 / _deg_of(degp_ref)
    z1_ref[...] = (sp_ref[0] + sp_ref[1] - z0_ref[...]) * dinv[:, None]


def _final(degp_ref, sp_ref, z1_ref, wt_ref, b_ref, out_ref):
    r = lax.rsqrt(_deg_of(degp_ref))
    x2 = (sp_ref[0] + sp_ref[1] - z1_ref[...]) * r[:, None]
    out_ref[...] = (
        jnp.dot(x2, wt_ref[...], preferred_element_type=jnp.float32)
        + b_ref[...]
    )


def kernel(x_indices, ei, emb_table, W, b):
    N, D = emb_table.shape
    OUT = W.shape[0]
    E = ei.shape[1]

    NP = (N // B + 1) * B
    assert NP % NW == 0 and NP % BLK == 0
    EPT = -(-E // (NW * B * NBUF)) * B * NBUF
    EP = NW * EPT
    NBE = EPT // B

    row_flat = jnp.concatenate([ei[0], jnp.zeros((EP - E,), jnp.int32)])
    col_flat = jnp.concatenate([ei[1], jnp.full((EP - E,), N, jnp.int32)])
    col3 = col_flat.reshape(NW, EPT // B, B)
    xind_pad = jnp.concatenate(
        [x_indices.astype(jnp.int32), jnp.zeros((NP - N,), jnp.int32)])
    ones8 = jnp.ones((B, WD), jnp.float32)
    zer8 = jnp.zeros((NP // NS, WD), jnp.float32)
    wt = W.T
    b2 = b.reshape(1, OUT)

    x, degp = _make_pass_a(N, D, NP, EPT)(
        emb_table, xind_pad, col_flat, ones8, zer8)

    grid = (NP // BLK,)
    degp_spec = pl.BlockSpec((NC, BLK, WD), lambda i: (0, i, 0))
    row_spec = pl.BlockSpec((BLK, D), lambda i: (i, 0))
    sp_spec = pl.BlockSpec((NC, BLK, D), lambda i: (0, i, 0))

    z0 = pl.pallas_call(
        _scale_z0,
        grid=grid,
        in_specs=[degp_spec, row_spec],
        out_specs=row_spec,
        out_shape=jax.ShapeDtypeStruct((NP, D), jnp.float32),
    )(degp, x)

    prop = _make_prop(D, NP, EPT)
    sp1 = prop(z0, row_flat, col3)

    z1 = pl.pallas_call(
        _combine_mid,
        grid=grid,
        in_specs=[degp_spec, sp_spec, row_spec],
        out_specs=row_spec,
        out_shape=jax.ShapeDtypeStruct((NP, D), jnp.float32),
    )(degp, sp1, z0)

    sp2 = prop(z1, row_flat, col3)

    out = pl.pallas_call(
        _final,
        grid=grid,
        in_specs=[
            degp_spec, sp_spec, row_spec,
            pl.BlockSpec((D, OUT), lambda i: (0, 0)),
            pl.BlockSpec((1, OUT), lambda i: (0, 0)),
        ],
        out_specs=pl.BlockSpec((BLK, OUT), lambda i: (i, 0)),
        out_shape=jax.ShapeDtypeStruct((NP, OUT), jnp.float32),
    )(degp, sp2, z1, wt, b2)

    return out[:N]


# speedup vs baseline: 1.0021x; 1.0021x over previous
import functools

import jax
import jax.numpy as jnp
from jax import lax
from jax.experimental import pallas as pl
from jax.experimental.pallas import tpu as pltpu
from jax.experimental.pallas import tpu_sc as plsc

NC = 2
NS = 16
NW = NC * NS
B = 128
BLK = 128

_MESH = functools.partial(
    plsc.VectorSubcoreMesh,
    core_axis_name="c", subcore_axis_name="s", num_cores=NC, num_subcores=NS,
)


def _worker_id():
    return lax.axis_index("c") * NS + lax.axis_index("s")


WD = 128


def _make_pass_a(N, D, NP, EPT):
    RPS = NP // NS
    NBX = NP // B
    TX = (NBX + NW - 1) // NW
    NBE = EPT // B

    NOUT = 8

    @functools.partial(
        pl.kernel,
        out_type=(
            jax.ShapeDtypeStruct((NP, D), jnp.float32),
            jax.ShapeDtypeStruct((NC, NP, WD), jnp.float32),
        ),
        mesh=_MESH(),
        scratch_types=[
            pltpu.VMEM((B,), jnp.int32),
            pltpu.VMEM((B, D), jnp.float32),
            pltpu.VMEM((B, WD), jnp.float32),
            pltpu.VMEM_SHARED((NP, WD), jnp.float32),
            pltpu.SemaphoreType.DMA,
        ],
    )
    def pass_a(emb_h, xind_h, col_h, ones_h, zer8_h, x_h, degp_h,
               idxv, rows, ones_v, deg_acc, sem):
        c = lax.axis_index("c")
        s = lax.axis_index("s")
        w = _worker_id()
        pltpu.sync_copy(zer8_h, deg_acc.at[pl.ds(s * RPS, RPS)])
        pltpu.sync_copy(ones_h, ones_v)
        plsc.subcore_barrier()

        def xgather(t, carry):
            bid = w + NW * t

            @pl.when(bid < NBX)
            def _():
                pltpu.sync_copy(xind_h.at[pl.ds(bid * B, B)], idxv)
                pltpu.async_copy(emb_h.at[idxv], rows, sem).wait()
                pltpu.sync_copy(rows, x_h.at[pl.ds(bid * B, B)])
            return carry

        lax.fori_loop(0, TX, xgather, 0)

        def deg_scatter(bi, carry):
            base = w * EPT + bi * B
            pltpu.sync_copy(col_h.at[pl.ds(base, B)], idxv)
            pltpu.sync_copy(ones_v, deg_acc.at[idxv], add=True)
            return carry

        lax.fori_loop(0, NBE, deg_scatter, 0)
        plsc.subcore_barrier()
        pltpu.sync_copy(deg_acc.at[pl.ds(s * RPS, RPS)],
                        degp_h.at[c, pl.ds(s * RPS, RPS)])

    return pass_a


NBUF = 2


def _make_prop(D, NP, EPT):
    RPS = NP // NS
    NBE = EPT // B
    assert NBE % 2 == 0
    NB2 = NBE // 2

    @functools.partial(
        pl.kernel,
        out_type=jax.ShapeDtypeStruct((NC, NP, D), jnp.float32),
        mesh=_MESH(),
        scratch_types=[
            pltpu.VMEM((NBE, B), jnp.int32),
            pltpu.VMEM((B,), jnp.int32),
            pltpu.VMEM((B,), jnp.int32),
            pltpu.VMEM((B, D), jnp.float32),
            pltpu.VMEM((B, D), jnp.float32),
            pltpu.VMEM_SHARED((NP, D), jnp.float32),
            pltpu.SemaphoreType.DMA,
            pltpu.SemaphoreType.DMA,
            pltpu.SemaphoreType.DMA,
        ],
    )
    def prop(z_h, row_h, col_h, sp_h, coli, rv0, rv1, rows0, rows1,
             acc, isem, gsem, ssem):
        c = lax.axis_index("c")
        s = lax.axis_index("s")
        w = _worker_id()
        rv = (rv0, rv1)
        rows = (rows0, rows1)
        pltpu.sync_copy(col_h.at[w], coli)
        pltpu.sync_copy(z_h.at[pl.ds(s * RPS, RPS)],
                        acc.at[pl.ds(s * RPS, RPS)])
        plsc.subcore_barrier()

        def fire_idx(i, p):
            pltpu.async_copy(row_h.at[pl.ds(w * EPT + i * B, B)], rv[p], isem)

        def wait_idx():
            pltpu.make_async_copy(row_h.at[pl.ds(0, B)], rv0, isem).wait()

        def fire_gather(p):
            pltpu.async_copy(z_h.at[rv[p]], rows[p], gsem)

        def wait_gather():
            pltpu.make_async_copy(z_h.at[rv0], rows0, gsem).wait()

        def fire_scatter(i, p):
            pltpu.async_copy(rows[p], acc.at[coli.at[i]], ssem, add=True)

        def wait_scatter():
            pltpu.make_async_copy(rows0, acc.at[coli.at[0]], ssem).wait()

        fire_idx(0, 0)
        fire_idx(1, 1)
        wait_idx()
        fire_gather(0)

        def pair(g, carry):
            for jj in range(2):
                i = 2 * g + jj
                wait_gather()
                fire_scatter(i, jj)

                @pl.when(i + 2 < NBE)
                def _():
                    fire_idx(i + 2, jj)

                @pl.when(i >= 1)
                def _():
                    wait_scatter()

                @pl.when(i + 1 < NBE)
                def _():
                    wait_idx()
                    fire_gather(1 - jj)
            return carry

        lax.fori_loop(0, NB2, pair, 0)
        wait_scatter()
        plsc.subcore_barrier()
        pltpu.sync_copy(acc.at[pl.ds(s * RPS, RPS)],
                        sp_h.at[c, pl.ds(s * RPS, RPS)])

    return prop


def _deg_of(degp_ref):
    return degp_ref[0, :, 0] + degp_ref[1, :, 0] + 1.0


def _scale_z0(degp_ref, x_ref, z_ref):
    r = lax.rsqrt(_deg_of(degp_ref))
    z_ref[...] = x_ref[...] * r[:, None]


def _combine_mid(degp_ref, sp_ref, z0_ref, z1_ref):
    dinv = 1.0 / _deg_of(degp_ref)
    z1_ref[...] = (sp_ref[0] + sp_ref[1] - z0_ref[...]) * dinv[:, None]


def _final(degp_ref, sp_ref, z1_ref, wt_ref, b_ref, out_ref):
    r = lax.rsqrt(_deg_of(degp_ref))
    x2 = (sp_ref[0] + sp_ref[1] - z1_ref[...]) * r[:, None]
    out_ref[...] = (
        jnp.dot(x2, wt_ref[...], preferred_element_type=jnp.float32)
        + b_ref[...]
    )


def kernel(x_indices, ei, emb_table, W, b):
    N, D = emb_table.shape
    OUT = W.shape[0]
    E = ei.shape[1]

    NP = (N // B + 1) * B
    assert NP % NW == 0 and NP % BLK == 0
    EPT = -(-E // (NW * B * NBUF)) * B * NBUF
    EP = NW * EPT
    NBE = EPT // B

    row_flat = jnp.concatenate([ei[0], jnp.zeros((EP - E,), jnp.int32)])
    col_flat = jnp.concatenate([ei[1], jnp.full((EP - E,), N, jnp.int32)])
    col3 = col_flat.reshape(NW, EPT // B, B)
    xind_pad = jnp.concatenate(
        [x_indices.astype(jnp.int32), jnp.zeros((NP - N,), jnp.int32)])
    ones8 = jnp.ones((B, WD), jnp.float32)
    zer8 = jnp.zeros((NP // NS, WD), jnp.float32)
    wt = W.T
    b2 = b.reshape(1, OUT)

    x, degp = _make_pass_a(N, D, NP, EPT)(
        emb_table, xind_pad, col_flat, ones8, zer8)

    grid = (NP // BLK,)
    degp_spec = pl.BlockSpec((NC, BLK, WD), lambda i: (0, i, 0))
    row_spec = pl.BlockSpec((BLK, D), lambda i: (i, 0))
    sp_spec = pl.BlockSpec((NC, BLK, D), lambda i: (0, i, 0))

    z0 = pl.pallas_call(
        _scale_z0,
        grid=grid,
        in_specs=[degp_spec, row_spec],
        out_specs=row_spec,
        out_shape=jax.ShapeDtypeStruct((NP, D), jnp.float32),
    )(degp, x)

    prop = _make_prop(D, NP, EPT)
    sp1 = prop(z0, row_flat, col3)

    z1 = pl.pallas_call(
        _combine_mid,
        grid=grid,
        in_specs=[degp_spec, sp_spec, row_spec],
        out_specs=row_spec,
        out_shape=jax.ShapeDtypeStruct((NP, D), jnp.float32),
    )(degp, sp1, z0)

    sp2 = prop(z1, row_flat, col3)

    out = pl.pallas_call(
        _final,
        grid=grid,
        in_specs=[
            degp_spec, sp_spec, row_spec,
            pl.BlockSpec((D, OUT), lambda i: (0, 0)),
            pl.BlockSpec((1, OUT), lambda i: (0, 0)),
        ],
        out_specs=pl.BlockSpec((BLK, OUT), lambda i: (i, 0)),
        out_shape=jax.ShapeDtypeStruct((NP, OUT), jnp.float32),
    )(degp, sp2, z1, wt, b2)

    return out[:N]


# speedup vs baseline: 1.0250x; 1.0229x over previous
import functools

import jax
import jax.numpy as jnp
from jax import lax
from jax.experimental import pallas as pl
from jax.experimental.pallas import tpu as pltpu
from jax.experimental.pallas import tpu_sc as plsc

NC = 2
NS = 16
NW = NC * NS
B = 128
BLK = 128

_MESH = functools.partial(
    plsc.VectorSubcoreMesh,
    core_axis_name="c", subcore_axis_name="s", num_cores=NC, num_subcores=NS,
)


def _worker_id():
    return lax.axis_index("c") * NS + lax.axis_index("s")


WD = 128


def _make_pass_a(N, D, NP, EPT):
    RPS = NP // NS
    NBX = NP // B
    TX = (NBX + NW - 1) // NW
    NBE = EPT // B

    NOUT = 8

    @functools.partial(
        pl.kernel,
        out_type=(
            jax.ShapeDtypeStruct((NP, D), jnp.float32),
            jax.ShapeDtypeStruct((NC, NP, WD), jnp.float32),
        ),
        mesh=_MESH(),
        scratch_types=[
            pltpu.VMEM((B,), jnp.int32),
            pltpu.VMEM((B, D), jnp.float32),
            pltpu.VMEM((B, WD), jnp.float32),
            pltpu.VMEM_SHARED((NP, WD), jnp.float32),
            pltpu.SemaphoreType.DMA,
        ],
    )
    def pass_a(emb_h, xind_h, col_h, ones_h, zer8_h, x_h, degp_h,
               idxv, rows, ones_v, deg_acc, sem):
        c = lax.axis_index("c")
        s = lax.axis_index("s")
        w = _worker_id()
        pltpu.sync_copy(zer8_h, deg_acc.at[pl.ds(s * RPS, RPS)])
        pltpu.sync_copy(ones_h, ones_v)
        plsc.subcore_barrier()

        def xgather(t, carry):
            bid = w + NW * t

            @pl.when(bid < NBX)
            def _():
                pltpu.sync_copy(xind_h.at[pl.ds(bid * B, B)], idxv)
                pltpu.async_copy(emb_h.at[idxv], rows, sem).wait()
                pltpu.sync_copy(rows, x_h.at[pl.ds(bid * B, B)])
            return carry

        lax.fori_loop(0, TX, xgather, 0)

        def deg_scatter(bi, carry):
            base = w * EPT + bi * B
            pltpu.sync_copy(col_h.at[pl.ds(base, B)], idxv)
            pltpu.sync_copy(ones_v, deg_acc.at[idxv], add=True)
            return carry

        lax.fori_loop(0, NBE, deg_scatter, 0)
        plsc.subcore_barrier()
        pltpu.sync_copy(deg_acc.at[pl.ds(s * RPS, RPS)],
                        degp_h.at[c, pl.ds(s * RPS, RPS)])

    return pass_a


RATIO = 0.76


def _make_prop(D, NP, NB0, NB1):
    RPS = NP // NS
    assert NB0 % 8 == 0 and NB1 % 8 == 0 and NB0 >= 8 and NB1 >= 8

    @functools.partial(
        pl.kernel,
        out_type=jax.ShapeDtypeStruct((NC, NP, D), jnp.float32),
        mesh=_MESH(),
        scratch_types=[
            pltpu.VMEM((NB0, B), jnp.int32),
            pltpu.VMEM((B,), jnp.int32),
            pltpu.VMEM((B,), jnp.int32),
            pltpu.VMEM((B, D), jnp.float32),
            pltpu.VMEM((B, D), jnp.float32),
            pltpu.VMEM_SHARED((NP, D), jnp.float32),
            pltpu.SemaphoreType.DMA,
            pltpu.SemaphoreType.DMA,
            pltpu.SemaphoreType.DMA,
        ],
    )
    def prop(z_h, row_h, col_h, sp_h, coli, rv0, rv1, rows0, rows1,
             acc, isem, gsem, ssem):
        c = lax.axis_index("c")
        s = lax.axis_index("s")
        nb = jnp.where(c == 0, NB0, NB1)
        bbase = pl.multiple_of(
            jnp.where(c == 0, s * NB0, NS * NB0 + s * NB1), 8)
        ebase = bbase * B
        rv = (rv0, rv1)
        rows = (rows0, rows1)
        pltpu.sync_copy(col_h.at[pl.ds(bbase, NB0)], coli)
        pltpu.sync_copy(z_h.at[pl.ds(s * RPS, RPS)],
                        acc.at[pl.ds(s * RPS, RPS)])
        plsc.subcore_barrier()

        def fire_idx(i, p):
            pltpu.async_copy(row_h.at[pl.ds(ebase + i * B, B)], rv[p], isem)

        def wait_idx():
            pltpu.make_async_copy(row_h.at[pl.ds(0, B)], rv0, isem).wait()

        def fire_gather(p):
            pltpu.async_copy(z_h.at[rv[p]], rows[p], gsem)

        def wait_gather():
            pltpu.make_async_copy(z_h.at[rv0], rows0, gsem).wait()

        def fire_scatter(i, p):
            pltpu.async_copy(rows[p], acc.at[coli.at[i]], ssem, add=True)

        def wait_scatter():
            pltpu.make_async_copy(rows0, acc.at[coli.at[0]], ssem).wait()

        fire_idx(0, 0)
        fire_idx(1, 1)
        wait_idx()
        fire_gather(0)

        def pair(g, carry):
            for jj in range(2):
                i = 2 * g + jj
                wait_gather()
                fire_scatter(i, jj)

                @pl.when(i + 2 < nb)
                def _():
                    fire_idx(i + 2, jj)

                @pl.when(i >= 1)
                def _():
                    wait_scatter()

                @pl.when(i + 1 < nb)
                def _():
                    wait_idx()
                    fire_gather(1 - jj)
            return carry

        lax.fori_loop(0, nb // 2, pair, 0)
        wait_scatter()
        plsc.subcore_barrier()
        pltpu.sync_copy(acc.at[pl.ds(s * RPS, RPS)],
                        sp_h.at[c, pl.ds(s * RPS, RPS)])

    return prop


def _deg_of(degp_ref):
    return degp_ref[0, :, 0] + degp_ref[1, :, 0] + 1.0


def _scale_z0(degp_ref, x_ref, z_ref):
    r = lax.rsqrt(_deg_of(degp_ref))
    z_ref[...] = x_ref[...] * r[:, None]


def _combine_mid(degp_ref, sp_ref, z0_ref, z1_ref):
    dinv = 1.0 / _deg_of(degp_ref)
    z1_ref[...] = (sp_ref[0] + sp_ref[1] - z0_ref[...]) * dinv[:, None]


def _final(degp_ref, sp_ref, z1_ref, wt_ref, b_ref, out_ref):
    r = lax.rsqrt(_deg_of(degp_ref))
    x2 = (sp_ref[0] + sp_ref[1] - z1_ref[...]) * r[:, None]
    out_ref[...] = (
        jnp.dot(x2, wt_ref[...], preferred_element_type=jnp.float32)
        + b_ref[...]
    )


def kernel(x_indices, ei, emb_table, W, b):
    N, D = emb_table.shape
    OUT = W.shape[0]
    E = ei.shape[1]

    NP = (N // B + 1) * B
    assert NP % NW == 0 and NP % BLK == 0
    TB = -(-E // (NS * B * 8)) * 8
    NB0 = min(TB - 8, max(8, 8 * int(round(TB * RATIO / 8))))
    NB1 = TB - NB0
    EPP = NS * TB * B
    EPA = EPP + max(NB0 - NB1, 0) * B

    row_flat = jnp.concatenate([ei[0], jnp.zeros((EPA - E,), jnp.int32)])
    col_flat = jnp.concatenate([ei[1], jnp.full((EPA - E,), N, jnp.int32)])
    col2 = col_flat.reshape(EPA // B, B)
    EPT_A = EPP // NW
    xind_pad = jnp.concatenate(
        [x_indices.astype(jnp.int32), jnp.zeros((NP - N,), jnp.int32)])
    ones8 = jnp.ones((B, WD), jnp.float32)
    zer8 = jnp.zeros((NP // NS, WD), jnp.float32)
    wt = W.T
    b2 = b.reshape(1, OUT)

    x, degp = _make_pass_a(N, D, NP, EPT_A)(
        emb_table, xind_pad, col_flat, ones8, zer8)

    grid = (NP // BLK,)
    degp_spec = pl.BlockSpec((NC, BLK, WD), lambda i: (0, i, 0))
    row_spec = pl.BlockSpec((BLK, D), lambda i: (i, 0))
    sp_spec = pl.BlockSpec((NC, BLK, D), lambda i: (0, i, 0))

    z0 = pl.pallas_call(
        _scale_z0,
        grid=grid,
        in_specs=[degp_spec, row_spec],
        out_specs=row_spec,
        out_shape=jax.ShapeDtypeStruct((NP, D), jnp.float32),
    )(degp, x)

    prop = _make_prop(D, NP, NB0, NB1)
    sp1 = prop(z0, row_flat, col2)

    z1 = pl.pallas_call(
        _combine_mid,
        grid=grid,
        in_specs=[degp_spec, sp_spec, row_spec],
        out_specs=row_spec,
        out_shape=jax.ShapeDtypeStruct((NP, D), jnp.float32),
    )(degp, sp1, z0)

    sp2 = prop(z1, row_flat, col2)

    out = pl.pallas_call(
        _final,
        grid=grid,
        in_specs=[
            degp_spec, sp_spec, row_spec,
            pl.BlockSpec((D, OUT), lambda i: (0, 0)),
            pl.BlockSpec((1, OUT), lambda i: (0, 0)),
        ],
        out_specs=pl.BlockSpec((BLK, OUT), lambda i: (i, 0)),
        out_shape=jax.ShapeDtypeStruct((NP, OUT), jnp.float32),
    )(degp, sp2, z1, wt, b2)

    return out[:N]


# speedup vs baseline: 1.2736x; 1.2425x over previous
import functools

import jax
import jax.numpy as jnp
from jax import lax
from jax.experimental import pallas as pl
from jax.experimental.pallas import tpu as pltpu
from jax.experimental.pallas import tpu_sc as plsc

NC = 2
NS = 16
NW = NC * NS
B = 128
BLK = 128

_MESH = functools.partial(
    plsc.VectorSubcoreMesh,
    core_axis_name="c", subcore_axis_name="s", num_cores=NC, num_subcores=NS,
)


def _worker_id():
    return lax.axis_index("c") * NS + lax.axis_index("s")


WD = 128


def _make_pass_a(N, D, NP, EPT):
    RPS = NP // NS
    NBX = NP // B
    TX = (NBX + NW - 1) // NW
    NBE = EPT // B

    NOUT = 8

    @functools.partial(
        pl.kernel,
        out_type=(
            jax.ShapeDtypeStruct((NP, D), jnp.float32),
            jax.ShapeDtypeStruct((NC, NP, WD), jnp.float32),
        ),
        mesh=_MESH(),
        scratch_types=[
            pltpu.VMEM((B,), jnp.int32),
            pltpu.VMEM((B, D), jnp.float32),
            pltpu.VMEM((B, WD), jnp.float32),
            pltpu.VMEM_SHARED((NP, WD), jnp.float32),
            pltpu.SemaphoreType.DMA,
        ],
    )
    def pass_a(emb_h, xind_h, col_h, ones_h, zer8_h, x_h, degp_h,
               idxv, rows, ones_v, deg_acc, sem):
        c = lax.axis_index("c")
        s = lax.axis_index("s")
        w = _worker_id()
        pltpu.sync_copy(zer8_h, deg_acc.at[pl.ds(s * RPS, RPS)])
        pltpu.sync_copy(ones_h, ones_v)
        plsc.subcore_barrier()

        def xgather(t, carry):
            bid = w + NW * t

            @pl.when(bid < NBX)
            def _():
                pltpu.sync_copy(xind_h.at[pl.ds(bid * B, B)], idxv)
                pltpu.async_copy(emb_h.at[idxv], rows, sem).wait()
                pltpu.sync_copy(rows, x_h.at[pl.ds(bid * B, B)])
            return carry

        lax.fori_loop(0, TX, xgather, 0)

        def deg_scatter(bi, carry):
            base = w * EPT + bi * B
            pltpu.sync_copy(col_h.at[pl.ds(base, B)], idxv)
            pltpu.sync_copy(ones_v, deg_acc.at[idxv], add=True)
            return carry

        lax.fori_loop(0, NBE, deg_scatter, 0)
        plsc.subcore_barrier()
        pltpu.sync_copy(deg_acc.at[pl.ds(s * RPS, RPS)],
                        degp_h.at[c, pl.ds(s * RPS, RPS)])

    return pass_a


def _make_prop(D, NP, NB):
    RPS = NP // NS
    assert NB % 2 == 0

    @functools.partial(
        pl.kernel,
        out_type=jax.ShapeDtypeStruct((NP, D), jnp.float32),
        mesh=_MESH(),
        scratch_types=[
            pltpu.VMEM((B,), jnp.int32),
            pltpu.VMEM((B,), jnp.int32),
            pltpu.VMEM((B,), jnp.int32),
            pltpu.VMEM((B,), jnp.int32),
            pltpu.VMEM((B, D), jnp.float32),
            pltpu.VMEM((B, D), jnp.float32),
            pltpu.VMEM_SHARED((NP, D), jnp.float32),
            pltpu.SemaphoreType.DMA,
            pltpu.SemaphoreType.DMA,
            pltpu.SemaphoreType.DMA,
            pltpu.SemaphoreType.DMA,
        ],
    )
    def prop(z_h, row_h, col_h, sp_h, rv0, rv1, cv0, cv1, rows0, rows1,
             acc, irsem, icsem, gsem, ssem):
        c = lax.axis_index("c")
        s = lax.axis_index("s")
        rv = (rv0, rv1)
        cv = (cv0, cv1)
        rows = (rows0, rows1)

        @pl.when(c == 0)
        def _run():
            ebase = s * NB * B

            def fire_ridx(i, p):
                pltpu.async_copy(
                    row_h.at[pl.ds(ebase + i * B, B)], rv[p], irsem)

            def wait_ridx():
                pltpu.make_async_copy(
                    row_h.at[pl.ds(0, B)], rv0, irsem).wait()

            def fire_cidx(i, p):
                pltpu.async_copy(
                    col_h.at[pl.ds(ebase + i * B, B)], cv[p], icsem)

            def wait_cidx():
                pltpu.make_async_copy(
                    col_h.at[pl.ds(0, B)], cv0, icsem).wait()

            def fire_gather(p):
                pltpu.async_copy(z_h.at[rv[p]], rows[p], gsem)

            def wait_gather():
                pltpu.make_async_copy(z_h.at[rv0], rows0, gsem).wait()

            def fire_scatter(p):
                pltpu.async_copy(rows[p], acc.at[cv[p]], ssem, add=True)

            def wait_scatter():
                pltpu.make_async_copy(rows0, acc.at[cv0], ssem).wait()

            pltpu.sync_copy(z_h.at[pl.ds(s * RPS, RPS)],
                            acc.at[pl.ds(s * RPS, RPS)])
            plsc.subcore_barrier()

            fire_ridx(0, 0)
            fire_ridx(1, 1)
            fire_cidx(0, 0)
            wait_ridx()
            fire_gather(0)

            def pair(g, carry):
                for jj in range(2):
                    i = 2 * g + jj
                    wait_gather()
                    wait_cidx()
                    fire_scatter(jj)

                    @pl.when(i + 2 < NB)
                    def _():
                        fire_ridx(i + 2, jj)

                    @pl.when(i >= 1)
                    def _():
                        wait_scatter()

                    @pl.when(i + 1 < NB)
                    def _():
                        fire_cidx(i + 1, 1 - jj)
                        wait_ridx()
                        fire_gather(1 - jj)
                return carry

            lax.fori_loop(0, NB // 2, pair, 0)
            wait_scatter()
            plsc.subcore_barrier()
            pltpu.sync_copy(acc.at[pl.ds(s * RPS, RPS)],
                            sp_h.at[pl.ds(s * RPS, RPS)])

    return prop


def _deg_of(degp_ref):
    return degp_ref[0, :, 0] + degp_ref[1, :, 0] + 1.0


def _scale_z0(degp_ref, x_ref, z_ref):
    r = lax.rsqrt(_deg_of(degp_ref))
    z_ref[...] = x_ref[...] * r[:, None]


def _combine_mid(degp_ref, sp_ref, z1_ref):
    dinv = 1.0 / _deg_of(degp_ref)
    z1_ref[...] = sp_ref[...] * dinv[:, None]


def _final(degp_ref, sp_ref, wt_ref, b_ref, out_ref):
    r = lax.rsqrt(_deg_of(degp_ref))
    x2 = sp_ref[...] * r[:, None]
    out_ref[...] = (
        jnp.dot(x2, wt_ref[...], preferred_element_type=jnp.float32)
        + b_ref[...]
    )


def kernel(x_indices, ei, emb_table, W, b):
    N, D = emb_table.shape
    OUT = W.shape[0]
    E = ei.shape[1]

    NP = (N // B + 1) * B
    assert NP % NW == 0 and NP % BLK == 0
    NB = -(-E // (NS * B * 2)) * 2
    EPP = NS * NB * B

    row_flat = jnp.concatenate([ei[0], jnp.zeros((EPP - E,), jnp.int32)])
    col_flat = jnp.concatenate([ei[1], jnp.full((EPP - E,), N, jnp.int32)])
    EPT_A = EPP // NW
    xind_pad = jnp.concatenate(
        [x_indices.astype(jnp.int32), jnp.zeros((NP - N,), jnp.int32)])
    ones8 = jnp.ones((B, WD), jnp.float32)
    zer8 = jnp.zeros((NP // NS, WD), jnp.float32)
    wt = W.T
    b2 = b.reshape(1, OUT)

    x, degp = _make_pass_a(N, D, NP, EPT_A)(
        emb_table, xind_pad, col_flat, ones8, zer8)

    grid = (NP // BLK,)
    degp_spec = pl.BlockSpec((NC, BLK, WD), lambda i: (0, i, 0))
    row_spec = pl.BlockSpec((BLK, D), lambda i: (i, 0))

    z0 = pl.pallas_call(
        _scale_z0,
        grid=grid,
        in_specs=[degp_spec, row_spec],
        out_specs=row_spec,
        out_shape=jax.ShapeDtypeStruct((NP, D), jnp.float32),
    )(degp, x)

    prop = _make_prop(D, NP, NB)
    sp1 = prop(z0, row_flat, col_flat)

    z1 = pl.pallas_call(
        _combine_mid,
        grid=grid,
        in_specs=[degp_spec, row_spec],
        out_specs=row_spec,
        out_shape=jax.ShapeDtypeStruct((NP, D), jnp.float32),
    )(degp, sp1)

    sp2 = prop(z1, row_flat, col_flat)

    out = pl.pallas_call(
        _final,
        grid=grid,
        in_specs=[
            degp_spec, row_spec,
            pl.BlockSpec((D, OUT), lambda i: (0, 0)),
            pl.BlockSpec((1, OUT), lambda i: (0, 0)),
        ],
        out_specs=pl.BlockSpec((BLK, OUT), lambda i: (i, 0)),
        out_shape=jax.ShapeDtypeStruct((NP, OUT), jnp.float32),
    )(degp, sp2, wt, b2)

    return out[:N]


# speedup vs baseline: 1.2788x; 1.0042x over previous
import functools

import jax
import jax.numpy as jnp
from jax import lax
from jax.experimental import pallas as pl
from jax.experimental.pallas import tpu as pltpu
from jax.experimental.pallas import tpu_sc as plsc

NC = 2
NS = 16
NW = NC * NS
B = 128
BLK = 1264

_MESH = functools.partial(
    plsc.VectorSubcoreMesh,
    core_axis_name="c", subcore_axis_name="s", num_cores=NC, num_subcores=NS,
)


def _worker_id():
    return lax.axis_index("c") * NS + lax.axis_index("s")


WD = 128


def _make_pass_a(N, D, NP, EPT):
    RPS = NP // NS
    NBX = NP // B
    TX = (NBX + NW - 1) // NW
    NBE = EPT // B

    NOUT = 8

    @functools.partial(
        pl.kernel,
        out_type=(
            jax.ShapeDtypeStruct((NP, D), jnp.float32),
            jax.ShapeDtypeStruct((NC, NP, WD), jnp.float32),
        ),
        mesh=_MESH(),
        scratch_types=[
            pltpu.VMEM((B,), jnp.int32),
            pltpu.VMEM((B, D), jnp.float32),
            pltpu.VMEM((B, WD), jnp.float32),
            pltpu.VMEM_SHARED((NP, WD), jnp.float32),
            pltpu.SemaphoreType.DMA,
        ],
    )
    def pass_a(emb_h, xind_h, col_h, ones_h, zer8_h, x_h, degp_h,
               idxv, rows, ones_v, deg_acc, sem):
        c = lax.axis_index("c")
        s = lax.axis_index("s")
        w = _worker_id()
        pltpu.sync_copy(zer8_h, deg_acc.at[pl.ds(s * RPS, RPS)])
        pltpu.sync_copy(ones_h, ones_v)
        plsc.subcore_barrier()

        def xgather(t, carry):
            bid = w + NW * t

            @pl.when(bid < NBX)
            def _():
                pltpu.sync_copy(xind_h.at[pl.ds(bid * B, B)], idxv)
                pltpu.async_copy(emb_h.at[idxv], rows, sem).wait()
                pltpu.sync_copy(rows, x_h.at[pl.ds(bid * B, B)])
            return carry

        lax.fori_loop(0, TX, xgather, 0)

        def deg_scatter(bi, carry):
            base = w * EPT + bi * B
            pltpu.sync_copy(col_h.at[pl.ds(base, B)], idxv)
            pltpu.sync_copy(ones_v, deg_acc.at[idxv], add=True)
            return carry

        lax.fori_loop(0, NBE, deg_scatter, 0)
        plsc.subcore_barrier()
        pltpu.sync_copy(deg_acc.at[pl.ds(s * RPS, RPS)],
                        degp_h.at[c, pl.ds(s * RPS, RPS)])

    return pass_a


def _make_prop(D, NP, NB):
    RPS = NP // NS
    assert NB % 2 == 0

    @functools.partial(
        pl.kernel,
        out_type=jax.ShapeDtypeStruct((NP, D), jnp.float32),
        mesh=_MESH(),
        scratch_types=[
            pltpu.VMEM((B,), jnp.int32),
            pltpu.VMEM((B,), jnp.int32),
            pltpu.VMEM((B,), jnp.int32),
            pltpu.VMEM((B,), jnp.int32),
            pltpu.VMEM((B, D), jnp.float32),
            pltpu.VMEM((B, D), jnp.float32),
            pltpu.VMEM_SHARED((NP, D), jnp.float32),
            pltpu.SemaphoreType.DMA,
            pltpu.SemaphoreType.DMA,
            pltpu.SemaphoreType.DMA,
            pltpu.SemaphoreType.DMA,
        ],
    )
    def prop(z_h, row_h, col_h, sp_h, rv0, rv1, cv0, cv1, rows0, rows1,
             acc, irsem, icsem, gsem, ssem):
        c = lax.axis_index("c")
        s = lax.axis_index("s")
        rv = (rv0, rv1)
        cv = (cv0, cv1)
        rows = (rows0, rows1)

        @pl.when(c == 0)
        def _run():
            ebase = s * NB * B

            def fire_ridx(i, p):
                pltpu.async_copy(
                    row_h.at[pl.ds(ebase + i * B, B)], rv[p], irsem)

            def wait_ridx():
                pltpu.make_async_copy(
                    row_h.at[pl.ds(0, B)], rv0, irsem).wait()

            def fire_cidx(i, p):
                pltpu.async_copy(
                    col_h.at[pl.ds(ebase + i * B, B)], cv[p], icsem)

            def wait_cidx():
                pltpu.make_async_copy(
                    col_h.at[pl.ds(0, B)], cv0, icsem).wait()

            def fire_gather(p):
                pltpu.async_copy(z_h.at[rv[p]], rows[p], gsem)

            def wait_gather():
                pltpu.make_async_copy(z_h.at[rv0], rows0, gsem).wait()

            def fire_scatter(p):
                pltpu.async_copy(rows[p], acc.at[cv[p]], ssem, add=True)

            def wait_scatter():
                pltpu.make_async_copy(rows0, acc.at[cv0], ssem).wait()

            pltpu.sync_copy(z_h.at[pl.ds(s * RPS, RPS)],
                            acc.at[pl.ds(s * RPS, RPS)])
            plsc.subcore_barrier()

            fire_ridx(0, 0)
            fire_ridx(1, 1)
            fire_cidx(0, 0)
            wait_ridx()
            fire_gather(0)

            def pair(g, carry):
                for jj in range(2):
                    i = 2 * g + jj
                    wait_gather()
                    wait_cidx()
                    fire_scatter(jj)

                    @pl.when(i + 2 < NB)
                    def _():
                        fire_ridx(i + 2, jj)

                    @pl.when(i >= 1)
                    def _():
                        wait_scatter()

                    @pl.when(i + 1 < NB)
                    def _():
                        fire_cidx(i + 1, 1 - jj)
                        wait_ridx()
                        fire_gather(1 - jj)
                return carry

            lax.fori_loop(0, NB // 2, pair, 0)
            wait_scatter()
            plsc.subcore_barrier()
            pltpu.sync_copy(acc.at[pl.ds(s * RPS, RPS)],
                            sp_h.at[pl.ds(s * RPS, RPS)])

    return prop


def _deg_of(degp_ref):
    return degp_ref[0, :, 0] + degp_ref[1, :, 0] + 1.0


def _scale_z0(degp_ref, x_ref, z_ref, r8_ref):
    r = lax.rsqrt(_deg_of(degp_ref))
    z_ref[...] = x_ref[...] * r[:, None]
    r8_ref[...] = r[:, None] * jnp.ones((1, 8), jnp.float32)


def _combine_mid(r8_ref, sp_ref, z1_ref):
    r = r8_ref[...][:, 0]
    z1_ref[...] = sp_ref[...] * (r * r)[:, None]


def _final(r8_ref, sp_ref, wt_ref, b_ref, out_ref):
    r = r8_ref[...][:, 0]
    x2 = sp_ref[...] * r[:, None]
    out_ref[...] = (
        jnp.dot(x2, wt_ref[...], preferred_element_type=jnp.float32)
        + b_ref[...]
    )


def kernel(x_indices, ei, emb_table, W, b):
    N, D = emb_table.shape
    OUT = W.shape[0]
    E = ei.shape[1]

    NP = (N // B + 1) * B
    assert NP % NW == 0 and NP % BLK == 0
    NB = -(-E // (NS * B * 2)) * 2
    EPP = NS * NB * B

    row_flat = jnp.concatenate([ei[0], jnp.zeros((EPP - E,), jnp.int32)])
    col_flat = jnp.concatenate([ei[1], jnp.full((EPP - E,), N, jnp.int32)])
    EPT_A = EPP // NW
    xind_pad = jnp.concatenate(
        [x_indices.astype(jnp.int32), jnp.zeros((NP - N,), jnp.int32)])
    ones8 = jnp.ones((B, WD), jnp.float32)
    zer8 = jnp.zeros((NP // NS, WD), jnp.float32)
    wt = W.T
    b2 = b.reshape(1, OUT)

    x, degp = _make_pass_a(N, D, NP, EPT_A)(
        emb_table, xind_pad, col_flat, ones8, zer8)

    grid = (NP // BLK,)
    degp_spec = pl.BlockSpec((NC, BLK, WD), lambda i: (0, i, 0))
    r8_spec = pl.BlockSpec((BLK, 8), lambda i: (i, 0))
    row_spec = pl.BlockSpec((BLK, D), lambda i: (i, 0))

    z0, r8 = pl.pallas_call(
        _scale_z0,
        grid=grid,
        in_specs=[degp_spec, row_spec],
        out_specs=[row_spec, r8_spec],
        out_shape=[jax.ShapeDtypeStruct((NP, D), jnp.float32),
                   jax.ShapeDtypeStruct((NP, 8), jnp.float32)],
    )(degp, x)

    prop = _make_prop(D, NP, NB)
    sp1 = prop(z0, row_flat, col_flat)

    z1 = pl.pallas_call(
        _combine_mid,
        grid=grid,
        in_specs=[r8_spec, row_spec],
        out_specs=row_spec,
        out_shape=jax.ShapeDtypeStruct((NP, D), jnp.float32),
    )(r8, sp1)

    sp2 = prop(z1, row_flat, col_flat)

    out = pl.pallas_call(
        _final,
        grid=grid,
        in_specs=[
            r8_spec, row_spec,
            pl.BlockSpec((D, OUT), lambda i: (0, 0)),
            pl.BlockSpec((1, OUT), lambda i: (0, 0)),
        ],
        out_specs=pl.BlockSpec((BLK, OUT), lambda i: (i, 0)),
        out_shape=jax.ShapeDtypeStruct((NP, OUT), jnp.float32),
    )(r8, sp2, wt, b2)

    return out[:N]
